# Initial kernel scaffold; baseline (speedup 1.0000x reference)
import jax
import jax.numpy as jnp
from jax.experimental import pallas as pl


def kernel(embeddings, target):
    raise NotImplementedError("write your pallas kernel here")


# speedup vs baseline: 2.6380x; 2.6380x over previous
import functools

import jax
import jax.numpy as jnp
from jax.experimental import pallas as pl

_N = 4096
_D = 64
_MARGIN = 1.0


def _triplet_block_kernel(bm, e_blk_ref, e_all_ref, t_row_ref, t_all_ref, out_ref):
    i = pl.program_id(0)
    e = e_blk_ref[...]
    ef = e_all_ref[...]
    ti = t_row_ref[...]
    tj = t_all_ref[...]

    sq_i = jnp.sum(e * e, axis=1, keepdims=True)
    sq_j = jnp.sum(ef * ef, axis=1, keepdims=True).T
    cross = jax.lax.dot_general(
        e, ef, (((1,), (1,)), ((), ())),
        preferred_element_type=jnp.float32)
    dist = jnp.maximum(sq_i + sq_j - 2.0 * cross, 0.0)

    same = ti.reshape(bm, 1) == tj
    row_glob = i * bm + jax.lax.broadcasted_iota(jnp.int32, (bm, _N), 0)
    col = jax.lax.broadcasted_iota(jnp.int32, (bm, _N), 1)
    eye = row_glob == col
    pos_mask = same & (~eye)
    neg_mask = ~same

    pos_v = jnp.max(jnp.where(pos_mask, dist, -1e9), axis=1)
    neg_v = jnp.min(jnp.where(neg_mask, dist, 1e9), axis=1)

    d0 = dist[:, 0]
    ap = jnp.where(pos_v > -1e8, pos_v, d0)
    an = jnp.where(neg_v < 1e8, neg_v, d0)

    losses = jnp.maximum(ap - an + _MARGIN, 0.0)
    out_ref[...] = jnp.sum(losses).reshape(1, 1, 1)


def _triplet_loss_sum(embeddings, target, bm):
    nb = _N // bm
    t2d = target.astype(jnp.int32).reshape(1, _N)
    partial = pl.pallas_call(
        functools.partial(_triplet_block_kernel, bm),
        grid=(nb,),
        in_specs=[
            pl.BlockSpec((bm, _D), lambda i: (i, 0)),
            pl.BlockSpec((_N, _D), lambda i: (0, 0)),
            pl.BlockSpec((1, bm), lambda i: (0, i)),
            pl.BlockSpec((1, _N), lambda i: (0, 0)),
        ],
        out_specs=pl.BlockSpec((1, 1, 1), lambda i: (i, 0, 0)),
        out_shape=jax.ShapeDtypeStruct((nb, 1, 1), jnp.float32),
    )(embeddings, embeddings, t2d, t2d)
    return jnp.sum(partial) / jnp.float32(_N)


def kernel(embeddings, target):
    mean_loss = _triplet_loss_sum(embeddings, target, bm=512)
    return (mean_loss, _N)
